# baseline (speedup 1.0000x reference)
import jax
import jax.numpy as jnp
from jax import lax
from jax.experimental import pallas as pl
from jax.experimental.pallas import tpu as pltpu, tpu_sc as plsc
import functools

DIM = 32
NC = 2
NS = 16
NW = NC * NS
SB = 128


@functools.partial(jax.jit, static_argnames=("n_sub",))
def _embed_call(ids3, table_r, table_i, n_sub):
    n = NW * n_sub * SB
    mesh = plsc.VectorSubcoreMesh(core_axis_name="c", subcore_axis_name="s")

    @functools.partial(
        pl.kernel,
        out_type=jax.ShapeDtypeStruct((n * 2 * DIM,), jnp.float32),
        mesh=mesh,
        compiler_params=pltpu.CompilerParams(
            needs_layout_passes=False, use_tc_tiling_on_sc=False
        ),
        scratch_types=[
            pltpu.VMEM((n_sub, SB), jnp.int32),
            pltpu.VMEM((SB, DIM), jnp.float32),
            pltpu.VMEM((SB, DIM), jnp.float32),
            pltpu.VMEM((SB * 2 * DIM,), jnp.float32),
            pltpu.SemaphoreType.DMA,
            pltpu.SemaphoreType.DMA,
        ],
    )
    def k(ids_hbm, tr_hbm, ti_hbm, out_hbm, idx_v, er_v, ei_v, out_v, sem_r, sem_i):
        wid = lax.axis_index("s") * NC + lax.axis_index("c")
        base = wid * (n_sub * SB)
        pltpu.sync_copy(ids_hbm.at[wid], idx_v)

        lane2 = lax.iota(jnp.int32, 16) * 2

        def interleave_row(r, _):
            cv = lane2 + r * (2 * DIM)
            plsc.store_scatter(out_v, [cv], er_v[r, pl.ds(0, 16)])
            plsc.store_scatter(out_v, [cv + 32], er_v[r, pl.ds(16, 16)])
            plsc.store_scatter(out_v, [cv + 1], ei_v[r, pl.ds(0, 16)])
            plsc.store_scatter(out_v, [cv + 33], ei_v[r, pl.ds(16, 16)])
            return 0

        def sub_batch(j, _):
            cp_r = pltpu.async_copy(tr_hbm.at[idx_v.at[j]], er_v, sem_r)
            cp_i = pltpu.async_copy(ti_hbm.at[idx_v.at[j]], ei_v, sem_i)
            cp_r.wait()
            cp_i.wait()
            lax.fori_loop(0, SB, interleave_row, 0, unroll=2)
            pltpu.sync_copy(
                out_v, out_hbm.at[pl.ds((base + j * SB) * 2 * DIM, SB * 2 * DIM)]
            )
            return 0

        lax.fori_loop(0, n_sub, sub_batch, 0)

    return k(ids3, table_r, table_i)


def kernel(input_ids, table_r, table_i):
    b, l = input_ids.shape
    n = b * l
    assert n % (NW * SB) == 0
    n_sub = n // (NW * SB)
    ids3 = input_ids.reshape(NW, n_sub, SB).astype(jnp.int32)
    out = _embed_call(ids3, table_r, table_i, n_sub)
    return out.reshape(b, l, DIM, 2)


# speedup vs baseline: 7.6493x; 7.6493x over previous
import functools

import jax
import jax.numpy as jnp
from jax import lax
from jax.experimental import pallas as pl
from jax.experimental.pallas import tpu as pltpu, tpu_sc as plsc

DIM = 32
NC = 2
NS = 16
NW = NC * NS
CH = 4096
BB = 128


def _pack_body(tr_ref, ti_ref, o_ref):
    rt = tr_ref[...].T
    it = ti_ref[...].T
    r3 = rt.reshape(CH // 2, 2, DIM)
    i3 = it.reshape(CH // 2, 2, DIM)
    o_ref[...] = jnp.concatenate(
        [r3[:, 0, :], i3[:, 0, :], r3[:, 1, :], i3[:, 1, :]], axis=1)


def _pack_tables(trp, tip):
    v = trp.shape[1]
    grid = (v + CH - 1) // CH
    return pl.pallas_call(
        _pack_body,
        grid=(grid,),
        in_specs=[
            pl.BlockSpec((DIM, CH), lambda i: (0, i)),
            pl.BlockSpec((DIM, CH), lambda i: (0, i)),
        ],
        out_specs=pl.BlockSpec((CH // 2, 4 * DIM), lambda i: (i, 0)),
        out_shape=jax.ShapeDtypeStruct((v // 2, 4 * DIM), jnp.float32),
    )(trp, tip)


@functools.partial(jax.jit, static_argnames=("n_l", "n_b"))
def _embed_call(ids_pad, table_r, table_i, n_l, n_b):
    n_lp = ids_pad.shape[0]
    n_bb = n_b // BB
    n_sb = (n_lp // 8) * n_bb
    k_per_w = n_sb // NW

    tab = _pack_tables(table_r.T, table_i.T)

    mesh = plsc.VectorSubcoreMesh(core_axis_name="c", subcore_axis_name="s")

    @functools.partial(
        pl.kernel,
        out_type=jax.ShapeDtypeStruct((n_l, DIM, n_bb, 2, BB), jnp.float32),
        mesh=mesh,
        compiler_params=pltpu.CompilerParams(needs_layout_passes=False),
        scratch_types=[
            pltpu.VMEM((8, BB), jnp.int32),
            pltpu.VMEM((BB,), jnp.int32),
            pltpu.VMEM((BB,), jnp.int32),
            pltpu.VMEM((BB, 4 * DIM), jnp.float32),
            pltpu.VMEM((DIM, 2, BB), jnp.float32),
            pltpu.SemaphoreType.DMA,
        ],
    )
    def k(ids_hbm, tab_hbm, out_hbm, idx_v, g_v, pc_v, gbuf, tbuf, sem):
        wid = lax.axis_index("s") * NC + lax.axis_index("c")
        lane = lax.iota(jnp.int32, 16)

        def do_l(l, bb):
            l8 = l % 8
            for t in range(8):
                v = idx_v[l8, pl.ds(16 * t, 16)]
                g_v[pl.ds(16 * t, 16)] = lax.shift_right_logical(v, 1)
                pc_v[pl.ds(16 * t, 16)] = lax.shift_left(v & 1, 6)
            pltpu.async_copy(tab_hbm.at[g_v], gbuf, sem).wait()
            for t in range(8):
                rows = lane + 16 * t
                pc = pc_v[pl.ds(16 * t, 16)]

                def body(d, _):
                    c0 = pc + d
                    tbuf[d, 0, pl.ds(16 * t, 16)] = plsc.load_gather(gbuf, [rows, c0])
                    tbuf[d, 1, pl.ds(16 * t, 16)] = plsc.load_gather(gbuf, [rows, c0 + DIM])
                    return 0

                lax.fori_loop(0, DIM, body, 0, unroll=4)

        def do_sb(kk, _):
            sb = kk * NW + wid
            lo = (sb // n_bb) * 8
            bb = sb % n_bb
            pltpu.sync_copy(
                ids_hbm.at[pl.ds(lo, 8), pl.ds(bb * BB, BB)], idx_v)

            def do_l8(l8, _):
                l = lo + l8

                @pl.when(l < n_l)
                def _():
                    do_l(l, bb)
                    pltpu.sync_copy(tbuf, out_hbm.at[l, :, bb, :, :])

                return 0

            lax.fori_loop(0, 8, do_l8, 0)
            return 0

        lax.fori_loop(0, k_per_w, do_sb, 0)

    return k(ids_pad, tab)


def kernel(input_ids, table_r, table_i):
    b, l = input_ids.shape
    n_lp = ((l + 7) // 8) * 8
    idsp = input_ids.astype(jnp.int32).T
    ids_pad = jnp.pad(idsp, ((0, n_lp - l), (0, 0)))
    out3 = _embed_call(ids_pad, table_r, table_i, l, b)
    t = jnp.transpose(out3, (2, 4, 0, 1, 3))
    return t.reshape(b, l, DIM, 2)


# speedup vs baseline: 8.9009x; 1.1636x over previous
import functools

import jax
import jax.numpy as jnp
from jax import lax
from jax.experimental import pallas as pl
from jax.experimental.pallas import tpu as pltpu, tpu_sc as plsc

DIM = 32
NC = 2
NS = 16
NW = NC * NS
CH = 4096
BB = 128


def _pack_body(tr_ref, ti_ref, o_ref):
    rt = tr_ref[...].T
    it = ti_ref[...].T
    r3 = rt.reshape(CH // 2, 2, DIM)
    i3 = it.reshape(CH // 2, 2, DIM)
    o_ref[...] = jnp.concatenate(
        [r3[:, 0, :], i3[:, 0, :], r3[:, 1, :], i3[:, 1, :]], axis=1)


def _pack_tables(trp, tip):
    v = trp.shape[1]
    grid = (v + CH - 1) // CH
    return pl.pallas_call(
        _pack_body,
        grid=(grid,),
        in_specs=[
            pl.BlockSpec((DIM, CH), lambda i: (0, i)),
            pl.BlockSpec((DIM, CH), lambda i: (0, i)),
        ],
        out_specs=pl.BlockSpec((CH // 2, 4 * DIM), lambda i: (i, 0)),
        out_shape=jax.ShapeDtypeStruct((v // 2, 4 * DIM), jnp.float32),
    )(trp, tip)


@functools.partial(jax.jit, static_argnames=("n_l", "n_b"))
def _embed_call(ids_pad, table_r, table_i, n_l, n_b):
    n_lp = ids_pad.shape[0]
    n_bb = n_b // BB
    n_sb = (n_lp // 8) * n_bb
    k_per_w = n_sb // NW

    tab = _pack_tables(table_r.T, table_i.T)

    mesh = plsc.VectorSubcoreMesh(core_axis_name="c", subcore_axis_name="s")

    l_tail = max(n_l % 8, 1)
    m_full = (n_l - n_l % 8) * n_bb // NW
    m_total = m_full + ((n_l % 8) * n_bb) // NW
    assert m_total % 2 == 0

    @functools.partial(
        pl.kernel,
        out_type=jax.ShapeDtypeStruct((n_l, DIM, n_bb, 2, BB), jnp.float32),
        mesh=mesh,
        compiler_params=pltpu.CompilerParams(needs_layout_passes=False),
        scratch_types=[
            pltpu.VMEM((8, BB), jnp.int32),
            pltpu.VMEM((2, BB), jnp.int32),
            pltpu.VMEM((2, BB), jnp.int32),
            pltpu.VMEM((2, BB, 4 * DIM), jnp.float32),
            pltpu.VMEM((2, DIM, 2, BB), jnp.float32),
            pltpu.SemaphoreType.DMA,
            pltpu.SemaphoreType.DMA,
            pltpu.SemaphoreType.DMA,
            pltpu.SemaphoreType.DMA,
        ],
    )
    def k(ids_hbm, tab_hbm, out_hbm, idx_v, g_v, pc_v, gbuf, tbuf,
          sem_g0, sem_g1, sem_w0, sem_w1):
        wid = lax.axis_index("s") * NC + lax.axis_index("c")
        lane = lax.iota(jnp.int32, 16)
        sem_g = (sem_g0, sem_g1)
        sem_w = (sem_w0, sem_w1)

        def coords(m):
            sbt = jnp.where(m < m_full, m // 8, m_full // 8 + (m - m_full) // l_tail)
            l8 = jnp.where(m < m_full, m % 8, (m - m_full) % l_tail)
            sb = sbt * NW + wid
            lo = (sb // n_bb) * 8
            bb = sb % n_bb
            return lo, l8, bb

        def prep(m, slot):
            lo, l8, bb = coords(m)

            @pl.when(l8 == 0)
            def _():
                pltpu.sync_copy(
                    ids_hbm.at[pl.ds(lo, 8), pl.ds(bb * BB, BB)], idx_v)

            for t in range(8):
                v = idx_v[l8, pl.ds(16 * t, 16)]
                g_v[slot, pl.ds(16 * t, 16)] = lax.shift_right_logical(v, 1)
                pc_v[slot, pl.ds(16 * t, 16)] = lax.shift_left(v & 1, 6)
            pltpu.async_copy(
                tab_hbm.at[g_v.at[slot]], gbuf.at[slot], sem_g[slot])

        def consume(m, slot):
            lo, l8, bb = coords(m)
            l = lo + l8
            @pl.when(m >= 2)
            def _():
                pltpu.make_async_copy(
                    tbuf.at[slot], out_hbm.at[0, :, 0, :, :], sem_w[slot]).wait()

            for t in range(8):
                rows = lane + 16 * t
                pc = pc_v[slot, pl.ds(16 * t, 16)]

                def body(d, _):
                    c0 = pc + d
                    tbuf[slot, d, 0, pl.ds(16 * t, 16)] = plsc.load_gather(
                        gbuf.at[slot], [rows, c0])
                    tbuf[slot, d, 1, pl.ds(16 * t, 16)] = plsc.load_gather(
                        gbuf.at[slot], [rows, c0 + DIM])
                    return 0

                lax.fori_loop(0, DIM, body, 0, unroll=4)
            pltpu.async_copy(
                tbuf.at[slot], out_hbm.at[l, :, bb, :, :], sem_w[slot])

        def wait_g(slot):
            pltpu.make_async_copy(
                tab_hbm.at[g_v.at[slot]], gbuf.at[slot], sem_g[slot]).wait()

        prep(0, 0)
        prep(1, 1)

        def body2(j, _):
            m0 = 2 * j
            wait_g(0)
            consume(m0, 0)

            @pl.when(m0 + 2 < m_total)
            def _():
                prep(m0 + 2, 0)

            wait_g(1)
            consume(m0 + 1, 1)

            @pl.when(m0 + 3 < m_total)
            def _():
                prep(m0 + 3, 1)

            return 0

        lax.fori_loop(0, m_total // 2, body2, 0)
        for slot in (0, 1):
            pltpu.make_async_copy(
                tbuf.at[slot], out_hbm.at[0, :, 0, :, :], sem_w[slot]).wait()

    return k(ids_pad, tab)


def kernel(input_ids, table_r, table_i):
    b, l = input_ids.shape
    n_lp = ((l + 7) // 8) * 8
    idsp = input_ids.astype(jnp.int32).T
    ids_pad = jnp.pad(idsp, ((0, n_lp - l), (0, 0)))
    out3 = _embed_call(ids_pad, table_r, table_i, l, b)
    t = jnp.transpose(out3, (2, 4, 0, 1, 3))
    return t.reshape(b, l, DIM, 2)


# speedup vs baseline: 10.7822x; 1.2114x over previous
import functools

import jax
import jax.numpy as jnp
from jax import lax
from jax.experimental import pallas as pl
from jax.experimental.pallas import tpu as pltpu, tpu_sc as plsc

DIM = 32
NC = 2
NS = 16
NW = NC * NS
CH = 4096
BB = 128


def _pack_body(ra_ref, rb_ref, ia_ref, ib_ref, o_ref):
    o_ref[...] = jnp.concatenate(
        [ra_ref[...].T, ia_ref[...].T, rb_ref[...].T, ib_ref[...].T], axis=1)


def _pack_tables(trp, tip):
    v = trp.shape[1]
    grid = (v + 2 * CH - 1) // (2 * CH)
    last = (v - 1) // CH

    def odd(i):
        return (0, jnp.minimum(2 * i + 1, last))

    return pl.pallas_call(
        _pack_body,
        grid=(grid,),
        in_specs=[
            pl.BlockSpec((DIM, CH), lambda i: (0, 2 * i)),
            pl.BlockSpec((DIM, CH), odd),
            pl.BlockSpec((DIM, CH), lambda i: (0, 2 * i)),
            pl.BlockSpec((DIM, CH), odd),
        ],
        out_specs=pl.BlockSpec((CH, 4 * DIM), lambda i: (i, 0)),
        out_shape=jax.ShapeDtypeStruct((grid * CH, 4 * DIM), jnp.float32),
    )(trp, trp, tip, tip)


@functools.partial(jax.jit, static_argnames=("n_l", "n_b"))
def _embed_call(ids_pad, table_r, table_i, n_l, n_b):
    n_lp = ids_pad.shape[0]
    n_bb = n_b // BB
    n_sb = (n_lp // 8) * n_bb
    k_per_w = n_sb // NW

    tab = _pack_tables(table_r.T, table_i.T)

    mesh = plsc.VectorSubcoreMesh(core_axis_name="c", subcore_axis_name="s")

    l_tail = max(n_l % 8, 1)
    m_full = (n_l - n_l % 8) * n_bb // NW
    m_total = m_full + ((n_l % 8) * n_bb) // NW
    assert m_total % 2 == 0

    @functools.partial(
        pl.kernel,
        out_type=jax.ShapeDtypeStruct((n_l, DIM, n_bb, 2, BB), jnp.float32),
        mesh=mesh,
        compiler_params=pltpu.CompilerParams(needs_layout_passes=False),
        scratch_types=[
            pltpu.VMEM((8, BB), jnp.int32),
            pltpu.VMEM((2, BB), jnp.int32),
            pltpu.VMEM((2, BB), jnp.int32),
            pltpu.VMEM((2, BB, 4 * DIM), jnp.float32),
            pltpu.VMEM((2, DIM, 2, BB), jnp.float32),
            pltpu.SemaphoreType.DMA,
            pltpu.SemaphoreType.DMA,
            pltpu.SemaphoreType.DMA,
            pltpu.SemaphoreType.DMA,
        ],
    )
    def k(ids_hbm, tab_hbm, out_hbm, idx_v, g_v, pc_v, gbuf, tbuf,
          sem_g0, sem_g1, sem_w0, sem_w1):
        wid = lax.axis_index("s") * NC + lax.axis_index("c")
        lane = lax.iota(jnp.int32, 16)
        sem_g = (sem_g0, sem_g1)
        sem_w = (sem_w0, sem_w1)

        def coords(m):
            sbt = jnp.where(m < m_full, m // 8, m_full // 8 + (m - m_full) // l_tail)
            l8 = jnp.where(m < m_full, m % 8, (m - m_full) % l_tail)
            sb = sbt * NW + wid
            lo = (sb // n_bb) * 8
            bb = sb % n_bb
            return lo, l8, bb

        def prep(m, slot):
            lo, l8, bb = coords(m)

            @pl.when(l8 == 0)
            def _():
                pltpu.sync_copy(
                    ids_hbm.at[pl.ds(lo, 8), pl.ds(bb * BB, BB)], idx_v)

            for t in range(8):
                v = idx_v[l8, pl.ds(16 * t, 16)]
                g_v[slot, pl.ds(16 * t, 16)] = (
                    lax.shift_left(lax.shift_right_logical(v, 13), 12)
                    | (v & (CH - 1)))
                pc_v[slot, pl.ds(16 * t, 16)] = lax.shift_left(
                    lax.shift_right_logical(v, 12) & 1, 6)
            pltpu.async_copy(
                tab_hbm.at[g_v.at[slot]], gbuf.at[slot], sem_g[slot])

        def consume(m, slot):
            lo, l8, bb = coords(m)
            l = lo + l8
            @pl.when(m >= 2)
            def _():
                pltpu.make_async_copy(
                    tbuf.at[slot], out_hbm.at[0, :, 0, :, :], sem_w[slot]).wait()

            for t in range(8):
                rows = lane + 16 * t
                pc = pc_v[slot, pl.ds(16 * t, 16)]

                def body(d, _):
                    c0 = pc + d
                    tbuf[slot, d, 0, pl.ds(16 * t, 16)] = plsc.load_gather(
                        gbuf.at[slot], [rows, c0])
                    tbuf[slot, d, 1, pl.ds(16 * t, 16)] = plsc.load_gather(
                        gbuf.at[slot], [rows, c0 + DIM])
                    return 0

                lax.fori_loop(0, DIM, body, 0, unroll=4)
            pltpu.async_copy(
                tbuf.at[slot], out_hbm.at[l, :, bb, :, :], sem_w[slot])

        def wait_g(slot):
            pltpu.make_async_copy(
                tab_hbm.at[g_v.at[slot]], gbuf.at[slot], sem_g[slot]).wait()

        prep(0, 0)
        prep(1, 1)

        def body2(j, _):
            m0 = 2 * j
            wait_g(0)
            consume(m0, 0)

            @pl.when(m0 + 2 < m_total)
            def _():
                prep(m0 + 2, 0)

            wait_g(1)
            consume(m0 + 1, 1)

            @pl.when(m0 + 3 < m_total)
            def _():
                prep(m0 + 3, 1)

            return 0

        lax.fori_loop(0, m_total // 2, body2, 0)
        for slot in (0, 1):
            pltpu.make_async_copy(
                tbuf.at[slot], out_hbm.at[0, :, 0, :, :], sem_w[slot]).wait()

    return k(ids_pad, tab)


def kernel(input_ids, table_r, table_i):
    b, l = input_ids.shape
    n_lp = ((l + 7) // 8) * 8
    idsp = input_ids.astype(jnp.int32).T
    ids_pad = jnp.pad(idsp, ((0, n_lp - l), (0, 0)))
    out3 = _embed_call(ids_pad, table_r, table_i, l, b)
    t = jnp.transpose(out3, (2, 4, 0, 1, 3))
    return t.reshape(b, l, DIM, 2)


# speedup vs baseline: 15.9171x; 1.4762x over previous
import functools

import jax
import jax.numpy as jnp
from jax import lax
from jax.experimental import pallas as pl
from jax.experimental.pallas import tpu as pltpu, tpu_sc as plsc

DIM = 32
NC = 2
NS = 16
NW = NC * NS
CH = 4096
BB = 128


def _pack_body(ra_ref, rb_ref, ia_ref, ib_ref, o_ref):
    o_ref[...] = jnp.concatenate(
        [ra_ref[...].T, ia_ref[...].T, rb_ref[...].T, ib_ref[...].T], axis=1)


def _pack_tables(trp, tip):
    v = trp.shape[1]
    grid = (v + 2 * CH - 1) // (2 * CH)
    last = (v - 1) // CH

    def odd(i):
        return (0, jnp.minimum(2 * i + 1, last))

    return pl.pallas_call(
        _pack_body,
        grid=(grid,),
        in_specs=[
            pl.BlockSpec((DIM, CH), lambda i: (0, 2 * i)),
            pl.BlockSpec((DIM, CH), odd),
            pl.BlockSpec((DIM, CH), lambda i: (0, 2 * i)),
            pl.BlockSpec((DIM, CH), odd),
        ],
        out_specs=pl.BlockSpec((CH, 4 * DIM), lambda i: (i, 0)),
        out_shape=jax.ShapeDtypeStruct((grid * CH, 4 * DIM), jnp.float32),
    )(trp, trp, tip, tip)


@functools.partial(jax.jit, static_argnames=("n_l", "n_b"))
def _embed_call(ids_pad, table_r, table_i, n_l, n_b):
    n_lp = ids_pad.shape[0]
    n_bb = n_b // BB
    n_sb = (n_lp // 8) * n_bb
    k_per_w = n_sb // NW

    tab = _pack_tables(table_r.T, table_i.T)

    mesh = plsc.VectorSubcoreMesh(core_axis_name="c", subcore_axis_name="s")

    l_tail = max(n_l % 8, 1)
    m_full = (n_l - n_l % 8) * n_bb // NW
    m_total = m_full + ((n_l % 8) * n_bb) // NW
    assert m_total % 2 == 0

    @functools.partial(
        pl.kernel,
        out_type=jax.ShapeDtypeStruct((n_l, DIM, n_bb, 2, BB), jnp.float32),
        mesh=mesh,
        compiler_params=pltpu.CompilerParams(needs_layout_passes=False),
        scratch_types=[
            pltpu.VMEM((8, BB), jnp.int32),
            pltpu.VMEM((2, BB), jnp.int32),
            pltpu.VMEM((2, BB), jnp.int32),
            pltpu.VMEM((2, BB, 4 * DIM), jnp.float32),
            pltpu.VMEM((2, DIM, 2, BB), jnp.float32),
            pltpu.SemaphoreType.DMA,
            pltpu.SemaphoreType.DMA,
            pltpu.SemaphoreType.DMA,
            pltpu.SemaphoreType.DMA,
        ],
    )
    def k(ids_hbm, tab_hbm, out_hbm, idx_v, g_v, pc_v, gbuf, tbuf,
          sem_g0, sem_g1, sem_w0, sem_w1):
        wid = lax.axis_index("s") * NC + lax.axis_index("c")
        lane = lax.iota(jnp.int32, 16)
        sem_g = (sem_g0, sem_g1)
        sem_w = (sem_w0, sem_w1)

        def coords(m):
            sbt = jnp.where(m < m_full, m // 8, m_full // 8 + (m - m_full) // l_tail)
            l8 = jnp.where(m < m_full, m % 8, (m - m_full) % l_tail)
            sb = sbt * NW + wid
            lo = (sb // n_bb) * 8
            bb = sb % n_bb
            return lo, l8, bb

        def prep(m, slot):
            lo, l8, bb = coords(m)

            @pl.when(l8 == 0)
            def _():
                pltpu.sync_copy(
                    ids_hbm.at[pl.ds(lo, 8), pl.ds(bb * BB, BB)], idx_v)

            for t in range(8):
                v = idx_v[l8, pl.ds(16 * t, 16)]
                g_v[slot, pl.ds(16 * t, 16)] = (
                    lax.shift_left(lax.shift_right_logical(v, 13), 12)
                    | (v & (CH - 1)))
                pc_v[slot, pl.ds(16 * t, 16)] = lax.shift_left(
                    lax.shift_right_logical(v, 12) & 1, 6)
            pltpu.async_copy(
                tab_hbm.at[g_v.at[slot]], gbuf.at[slot], sem_g[slot])

        def consume(m, slot):
            lo, l8, bb = coords(m)
            l = lo + l8
            @pl.when(m >= 2)
            def _():
                pltpu.make_async_copy(
                    tbuf.at[slot], out_hbm.at[0, :, 0, :, :], sem_w[slot]).wait()

            for t in range(8):
                rows = lane + 16 * t
                pc = pc_v[slot, pl.ds(16 * t, 16)]

                @plsc.parallel_loop(0, DIM, unroll=4)
                def _(d):
                    c0 = pc + d
                    tbuf[slot, d, 0, pl.ds(16 * t, 16)] = plsc.load_gather(
                        gbuf.at[slot], [rows, c0])
                    tbuf[slot, d, 1, pl.ds(16 * t, 16)] = plsc.load_gather(
                        gbuf.at[slot], [rows, c0 + DIM])
            pltpu.async_copy(
                tbuf.at[slot], out_hbm.at[l, :, bb, :, :], sem_w[slot])

        def wait_g(slot):
            pltpu.make_async_copy(
                tab_hbm.at[g_v.at[slot]], gbuf.at[slot], sem_g[slot]).wait()

        prep(0, 0)
        prep(1, 1)

        def body2(j, _):
            m0 = 2 * j
            wait_g(0)
            consume(m0, 0)

            @pl.when(m0 + 2 < m_total)
            def _():
                prep(m0 + 2, 0)

            wait_g(1)
            consume(m0 + 1, 1)

            @pl.when(m0 + 3 < m_total)
            def _():
                prep(m0 + 3, 1)

            return 0

        lax.fori_loop(0, m_total // 2, body2, 0)
        for slot in (0, 1):
            pltpu.make_async_copy(
                tbuf.at[slot], out_hbm.at[0, :, 0, :, :], sem_w[slot]).wait()

    return k(ids_pad, tab)


def kernel(input_ids, table_r, table_i):
    b, l = input_ids.shape
    n_lp = ((l + 7) // 8) * 8
    idsp = input_ids.astype(jnp.int32).T
    ids_pad = jnp.pad(idsp, ((0, n_lp - l), (0, 0)))
    out3 = _embed_call(ids_pad, table_r, table_i, l, b)
    t = jnp.transpose(out3, (2, 4, 0, 1, 3))
    return t.reshape(b, l, DIM, 2)


# speedup vs baseline: 15.9878x; 1.0044x over previous
import functools

import jax
import jax.numpy as jnp
from jax import lax
from jax.experimental import pallas as pl
from jax.experimental.pallas import tpu as pltpu, tpu_sc as plsc

DIM = 32
NC = 2
NS = 16
NW = NC * NS
CH = 4096
BB = 128


def _pack_body(ra_ref, rb_ref, ia_ref, ib_ref, o_ref):
    o_ref[...] = jnp.concatenate(
        [ra_ref[...].T, ia_ref[...].T, rb_ref[...].T, ib_ref[...].T], axis=1)


def _pack_tables(trp, tip):
    v = trp.shape[1]
    grid = (v + 2 * CH - 1) // (2 * CH)
    last = (v - 1) // CH

    def odd(i):
        return (0, jnp.minimum(2 * i + 1, last))

    return pl.pallas_call(
        _pack_body,
        grid=(grid,),
        in_specs=[
            pl.BlockSpec((DIM, CH), lambda i: (0, 2 * i)),
            pl.BlockSpec((DIM, CH), odd),
            pl.BlockSpec((DIM, CH), lambda i: (0, 2 * i)),
            pl.BlockSpec((DIM, CH), odd),
        ],
        out_specs=pl.BlockSpec((CH, 4 * DIM), lambda i: (i, 0)),
        out_shape=jax.ShapeDtypeStruct((grid * CH, 4 * DIM), jnp.float32),
    )(trp, trp, tip, tip)


@functools.partial(jax.jit, static_argnames=("n_l", "n_b"))
def _embed_call(ids_pad, table_r, table_i, n_l, n_b):
    n_lp = ids_pad.shape[0]
    n_bb = n_b // BB
    n_sb = (n_lp // 8) * n_bb
    k_per_w = n_sb // NW

    tab = _pack_tables(table_r.T, table_i.T)
    tab64 = tab.reshape(tab.shape[0] * 2, 2 * DIM)

    mesh = plsc.VectorSubcoreMesh(core_axis_name="c", subcore_axis_name="s")

    l_tail = max(n_l % 8, 1)
    m_full = (n_l - n_l % 8) * n_bb // NW
    m_total = m_full + ((n_l % 8) * n_bb) // NW
    assert m_total % 2 == 0

    @functools.partial(
        pl.kernel,
        out_type=jax.ShapeDtypeStruct((n_l, DIM, n_bb, 2, BB), jnp.float32),
        mesh=mesh,
        compiler_params=pltpu.CompilerParams(
            needs_layout_passes=False, use_tc_tiling_on_sc=False),
        scratch_types=[
            pltpu.VMEM((8, BB), jnp.int32),
            pltpu.VMEM((2, BB), jnp.int32),
            pltpu.VMEM((2, BB, 2 * DIM), jnp.float32),
            pltpu.VMEM((2, DIM, 2, BB), jnp.float32),
            pltpu.SemaphoreType.DMA,
            pltpu.SemaphoreType.DMA,
            pltpu.SemaphoreType.DMA,
            pltpu.SemaphoreType.DMA,
        ],
    )
    def k(ids_hbm, tab_hbm, out_hbm, idx_v, g_v, gbuf, tbuf,
          sem_g0, sem_g1, sem_w0, sem_w1):
        wid = lax.axis_index("s") * NC + lax.axis_index("c")
        lane = lax.iota(jnp.int32, 16)
        sem_g = (sem_g0, sem_g1)
        sem_w = (sem_w0, sem_w1)

        def coords(m):
            sbt = jnp.where(m < m_full, m // 8, m_full // 8 + (m - m_full) // l_tail)
            l8 = jnp.where(m < m_full, m % 8, (m - m_full) % l_tail)
            sb = sbt * NW + wid
            lo = (sb // n_bb) * 8
            bb = sb % n_bb
            return lo, l8, bb

        def prep(m, slot):
            lo, l8, bb = coords(m)

            @pl.when(l8 == 0)
            def _():
                pltpu.sync_copy(
                    ids_hbm.at[pl.ds(lo, 8), pl.ds(bb * BB, BB)], idx_v)

            for t in range(8):
                v = idx_v[l8, pl.ds(16 * t, 16)]
                g_v[slot, pl.ds(16 * t, 16)] = (
                    lax.shift_left(lax.shift_right_logical(v, 13), 13)
                    | lax.shift_left(v & (CH - 1), 1)
                    | (lax.shift_right_logical(v, 12) & 1))
            pltpu.async_copy(
                tab_hbm.at[g_v.at[slot]], gbuf.at[slot], sem_g[slot])

        def consume(m, slot):
            lo, l8, bb = coords(m)
            l = lo + l8
            @pl.when(m >= 2)
            def _():
                pltpu.make_async_copy(
                    tbuf.at[slot], out_hbm.at[0, :, 0, :, :], sem_w[slot]).wait()

            zero = jnp.zeros((16,), jnp.int32)
            for t in range(8):
                rows = lane + 16 * t

                @plsc.parallel_loop(0, DIM, unroll=4)
                def _(d):
                    c0 = zero + d
                    tbuf[slot, d, 0, pl.ds(16 * t, 16)] = plsc.load_gather(
                        gbuf.at[slot], [rows, c0])
                    tbuf[slot, d, 1, pl.ds(16 * t, 16)] = plsc.load_gather(
                        gbuf.at[slot], [rows, c0 + DIM])
            pltpu.async_copy(
                tbuf.at[slot], out_hbm.at[l, :, bb, :, :], sem_w[slot])

        def wait_g(slot):
            pltpu.make_async_copy(
                tab_hbm.at[g_v.at[slot]], gbuf.at[slot], sem_g[slot]).wait()

        prep(0, 0)
        prep(1, 1)

        def body2(j, _):
            m0 = 2 * j
            wait_g(0)
            consume(m0, 0)

            @pl.when(m0 + 2 < m_total)
            def _():
                prep(m0 + 2, 0)

            wait_g(1)
            consume(m0 + 1, 1)

            @pl.when(m0 + 3 < m_total)
            def _():
                prep(m0 + 3, 1)

            return 0

        lax.fori_loop(0, m_total // 2, body2, 0)
        for slot in (0, 1):
            pltpu.make_async_copy(
                tbuf.at[slot], out_hbm.at[0, :, 0, :, :], sem_w[slot]).wait()

    return k(ids_pad, tab64)


def kernel(input_ids, table_r, table_i):
    b, l = input_ids.shape
    n_lp = ((l + 7) // 8) * 8
    idsp = input_ids.astype(jnp.int32).T
    ids_pad = jnp.pad(idsp, ((0, n_lp - l), (0, 0)))
    out3 = _embed_call(ids_pad, table_r, table_i, l, b)
    t = jnp.transpose(out3, (2, 4, 0, 1, 3))
    return t.reshape(b, l, DIM, 2)
